# baseline (device time: 6397 ns/iter reference)
import jax
import jax.numpy as jnp
from jax import lax
from jax.experimental import pallas as pl
from jax.experimental.pallas import tpu as pltpu

Y_SIZE = 2


def kernel(x, gamma):
    m, n = x.shape
    n_global = n * Y_SIZE
    eps = 1e-5
    H = m // 2
    T = H // 128

    def body(x_ref, g_ref, out_ref, psum_ref, recv_ref, send_sems, recv_sems):
        my_x = lax.axis_index("x")
        my_y = lax.axis_index("y")
        nbr = (my_x, 1 - my_y)

        barrier_sem = pltpu.get_barrier_semaphore()
        pl.semaphore_signal(
            barrier_sem, inc=1, device_id=nbr,
            device_id_type=pl.DeviceIdType.MESH,
        )

        g16 = g_ref[...].reshape(1, 1, n).astype(jnp.bfloat16)

        def xhalf(i):
            return x_ref[pl.ds(i * H, H), :].reshape(T, 128, n)

        rdmas = []
        for i in range(2):
            xh = xhalf(i)
            psum_ref[i] = jnp.sum(xh * xh, axis=2)
            if i == 0:
                pl.semaphore_wait(barrier_sem, 1)
            rdma = pltpu.make_async_remote_copy(
                src_ref=psum_ref.at[i],
                dst_ref=recv_ref.at[i],
                send_sem=send_sems.at[i],
                recv_sem=recv_sems.at[i],
                device_id=nbr,
                device_id_type=pl.DeviceIdType.MESH,
            )
            rdma.start()
            rdmas.append(rdma)

        for i in range(2):
            xg = xhalf(i).astype(jnp.bfloat16) * g16
            rdmas[i].wait_recv()
            total = psum_ref[i] + recv_ref[i]
            inv16 = lax.rsqrt(total * (1.0 / n_global) + eps).astype(jnp.bfloat16)
            out_ref[pl.ds(i * H, H), :] = (xg * inv16[:, :, None]).reshape(H, n)

        for r in rdmas:
            r.wait_send()

    return pl.pallas_call(
        body,
        out_shape=jax.ShapeDtypeStruct((m, n), jnp.bfloat16),
        in_specs=[
            pl.BlockSpec(memory_space=pltpu.VMEM),
            pl.BlockSpec(memory_space=pltpu.VMEM),
        ],
        out_specs=pl.BlockSpec(memory_space=pltpu.VMEM),
        scratch_shapes=[
            pltpu.VMEM((2, T, 128), jnp.float32),
            pltpu.VMEM((2, T, 128), jnp.float32),
            pltpu.SemaphoreType.DMA((2,)),
            pltpu.SemaphoreType.DMA((2,)),
        ],
        compiler_params=pltpu.CompilerParams(collective_id=0),
    )(x, gamma.reshape(1, n))


# device time: 6324 ns/iter; 1.0115x vs baseline; 1.0115x over previous
import jax
import jax.numpy as jnp
from jax import lax
from jax.experimental import pallas as pl
from jax.experimental.pallas import tpu as pltpu

Y_SIZE = 2


def kernel(x, gamma):
    m, n = x.shape
    n_global = n * Y_SIZE
    eps = 1e-5

    def body(x_ref, g_ref, out_ref, psum_ref, recv_ref, send_sem, recv_sem):
        my_x = lax.axis_index("x")
        my_y = lax.axis_index("y")
        nbr = (my_x, 1 - my_y)

        barrier_sem = pltpu.get_barrier_semaphore()
        pl.semaphore_signal(
            barrier_sem, inc=1, device_id=nbr,
            device_id_type=pl.DeviceIdType.MESH,
        )

        xr = x_ref[...].reshape(m // 128, 128, n)
        psum_ref[...] = jnp.sum(xr * xr, axis=2)

        pl.semaphore_wait(barrier_sem, 1)

        rdma = pltpu.make_async_remote_copy(
            src_ref=psum_ref,
            dst_ref=recv_ref,
            send_sem=send_sem,
            recv_sem=recv_sem,
            device_id=nbr,
            device_id_type=pl.DeviceIdType.MESH,
        )
        rdma.start()

        xg = xr.astype(jnp.bfloat16) * g_ref[...].reshape(1, 1, n).astype(jnp.bfloat16)

        rdma.wait_recv()

        total = psum_ref[...] + recv_ref[...]
        inv = lax.rsqrt(total * (1.0 / n_global) + eps)
        inv16 = inv.astype(jnp.bfloat16)
        out_ref[...] = (xg * inv16[:, :, None]).reshape(m, n)

        rdma.wait_send()

    return pl.pallas_call(
        body,
        out_shape=jax.ShapeDtypeStruct((m, n), jnp.bfloat16),
        in_specs=[
            pl.BlockSpec(memory_space=pltpu.VMEM),
            pl.BlockSpec(memory_space=pltpu.VMEM),
        ],
        out_specs=pl.BlockSpec(memory_space=pltpu.VMEM),
        scratch_shapes=[
            pltpu.VMEM((m // 128, 128), jnp.float32),
            pltpu.VMEM((m // 128, 128), jnp.float32),
            pltpu.SemaphoreType.DMA,
            pltpu.SemaphoreType.DMA,
        ],
        compiler_params=pltpu.CompilerParams(collective_id=0),
    )(x, gamma.reshape(1, n))
